# Initial kernel scaffold; baseline (speedup 1.0000x reference)
import jax
import jax.numpy as jnp
from jax.experimental import pallas as pl


def kernel(unary, ref):
    raise NotImplementedError("write your pallas kernel here")


# speedup vs baseline: 3.9897x; 3.9897x over previous
import functools

import jax
import jax.numpy as jnp
import numpy as np
from jax.experimental import pallas as pl

_SXY_BF = 70.0
_SC_BF = 12.0
_COMPAT_BF = 4.0
_SXY_SPATIAL = 3
_COMPAT_SPATIAL = 2.0
_NUM_ITER = 2

_H = 64
_W = 64
_HW = _H * _W
_C = 21
_N = 2

_I_BLK = 256
_J_BLK = 1024


def _spatial_matrix():
    sig_sq = float(_SXY_SPATIAL ** 2)
    r = int(sig_sq if sig_sq % 2 else sig_sq - 1)
    s = 2 * r + 1
    g1 = np.exp(-((np.arange(s, dtype=np.float64) - r) ** 2) / (2.0 * sig_sq))
    g1 = g1 / g1.sum()
    a = np.zeros((_H, _H), dtype=np.float64)
    for y in range(_H):
        lo = max(0, y - r)
        hi = min(_H, y + r + 1)
        a[y, lo:hi] = g1[(lo - y + r):(hi - y + r)]
    return jnp.asarray(a, dtype=jnp.float32)


def _build_kern(fa_ref, fb_ref, un_ref, k_ref, inv_ref, u_ref, q0_ref):
    fa = fa_ref[0]
    fb = fb_ref[0]
    e = jax.lax.dot_general(fa, fb, (((1,), (1,)), ((), ())),
                            preferred_element_type=jnp.float32)
    k = jnp.exp(jnp.minimum(e, 0.0))
    k_ref[0] = k
    rs = jnp.sum(k, axis=1)
    inv_ref[0, 0] = 1.0 / (jnp.sqrt(rs) + 1e-8)
    u = jnp.log(jnp.clip(un_ref[0], 1e-5, 1.0))
    u_ref[0] = u
    m = jnp.max(u, axis=0, keepdims=True)
    ex = jnp.exp(u - m)
    q0_ref[0] = ex / jnp.sum(ex, axis=0, keepdims=True)


def _sconv_kern(q_ref, a_ref, out_ref):
    q = q_ref[0]
    a = a_ref[...]
    s1 = jax.lax.dot_general(q, a, (((1,), (0,)), ((), ())),
                             preferred_element_type=jnp.float32)
    s2 = jax.lax.dot_general(s1, a, (((1,), (0,)), ((), ())),
                             preferred_element_type=jnp.float32)
    out_ref[0] = s2


def _step_kern(q_ref, inv_ref, k_ref, invj_ref, u_ref, qsf_ref, out_ref):
    v = q_ref[0] * inv_ref[0]
    acc = jax.lax.dot_general(v, k_ref[0], (((1,), (0,)), ((), ())),
                              preferred_element_type=jnp.float32)
    qbf = acc * invj_ref[0]
    qh = u_ref[0] + _COMPAT_BF * qbf + _COMPAT_SPATIAL * qsf_ref[0]
    m = jnp.max(qh, axis=0, keepdims=True)
    ex = jnp.exp(qh - m)
    out_ref[0] = ex / jnp.sum(ex, axis=0, keepdims=True)


@jax.jit
def kernel(unary, ref):
    n, c, h, w = unary.shape
    hw = h * w
    kstd = jnp.array([_SXY_BF, _SXY_BF, _SC_BF, _SC_BF, _SC_BF],
                     dtype=jnp.float32)
    y = jax.lax.broadcasted_iota(jnp.float32, (h, w), 0)
    x = jax.lax.broadcasted_iota(jnp.float32, (h, w), 1)
    yx = jnp.broadcast_to(jnp.stack([y, x], 0)[None], (n, 2, h, w))
    f = jnp.concatenate([yx, ref], axis=1) / kstd[None, :, None, None]
    f = f.reshape(n, 5, hw)
    f2 = jnp.sum(f * f, axis=1, keepdims=True)
    ones = jnp.ones_like(f2)
    zeros = jnp.zeros_like(f2)
    fa = jnp.concatenate([f, -0.5 * f2, ones, zeros], axis=1)
    fb = jnp.concatenate([f, ones, -0.5 * f2, zeros], axis=1)
    fa = jnp.transpose(fa, (0, 2, 1))
    fb = jnp.transpose(fb, (0, 2, 1))
    un = unary.reshape(n, c, hw)

    n_i = hw // _I_BLK
    kmat, inv_norm, u, q = pl.pallas_call(
        _build_kern,
        grid=(n, n_i),
        in_specs=[
            pl.BlockSpec((1, _I_BLK, 8), lambda b, i: (b, i, 0)),
            pl.BlockSpec((1, hw, 8), lambda b, i: (b, 0, 0)),
            pl.BlockSpec((1, c, _I_BLK), lambda b, i: (b, 0, i)),
        ],
        out_specs=[
            pl.BlockSpec((1, _I_BLK, hw), lambda b, i: (b, i, 0)),
            pl.BlockSpec((1, 1, _I_BLK), lambda b, i: (b, 0, i)),
            pl.BlockSpec((1, c, _I_BLK), lambda b, i: (b, 0, i)),
            pl.BlockSpec((1, c, _I_BLK), lambda b, i: (b, 0, i)),
        ],
        out_shape=[
            jax.ShapeDtypeStruct((n, hw, hw), jnp.float32),
            jax.ShapeDtypeStruct((n, 1, hw), jnp.float32),
            jax.ShapeDtypeStruct((n, c, hw), jnp.float32),
            jax.ShapeDtypeStruct((n, c, hw), jnp.float32),
        ],
    )(fa, fb, un)

    a = _spatial_matrix()
    sconv = pl.pallas_call(
        _sconv_kern,
        grid=(n,),
        in_specs=[
            pl.BlockSpec((1, c, h, w), lambda b: (b, 0, 0, 0)),
            pl.BlockSpec((h, h), lambda b: (0, 0)),
        ],
        out_specs=pl.BlockSpec((1, c, h, w), lambda b: (b, 0, 0, 0)),
        out_shape=jax.ShapeDtypeStruct((n, c, h, w), jnp.float32),
    )

    n_j = hw // _J_BLK
    step = pl.pallas_call(
        _step_kern,
        grid=(n, n_j),
        in_specs=[
            pl.BlockSpec((1, c, hw), lambda b, j: (b, 0, 0)),
            pl.BlockSpec((1, 1, hw), lambda b, j: (b, 0, 0)),
            pl.BlockSpec((1, hw, _J_BLK), lambda b, j: (b, 0, j)),
            pl.BlockSpec((1, 1, _J_BLK), lambda b, j: (b, 0, j)),
            pl.BlockSpec((1, c, _J_BLK), lambda b, j: (b, 0, j)),
            pl.BlockSpec((1, c, _J_BLK), lambda b, j: (b, 0, j)),
        ],
        out_specs=pl.BlockSpec((1, c, _J_BLK), lambda b, j: (b, 0, j)),
        out_shape=jax.ShapeDtypeStruct((n, c, hw), jnp.float32),
    )

    for _ in range(_NUM_ITER):
        qsf = sconv(q.reshape(n, c, h, w), a).reshape(n, c, hw)
        q = step(q, inv_norm, kmat, inv_norm, u, qsf)
    return q.reshape(n, c, h, w)


# speedup vs baseline: 5.2992x; 1.3282x over previous
import functools

import jax
import jax.numpy as jnp
import numpy as np
from jax.experimental import pallas as pl

_SXY_BF = 70.0
_SC_BF = 12.0
_COMPAT_BF = 4.0
_SXY_SPATIAL = 3
_COMPAT_SPATIAL = 2.0
_NUM_ITER = 2

_H = 64
_W = 64
_HW = _H * _W
_C = 21
_N = 2

_I_BLK = 256
_J_BLK = 1024


def _spatial_matrix():
    sig_sq = float(_SXY_SPATIAL ** 2)
    r = int(sig_sq if sig_sq % 2 else sig_sq - 1)
    s = 2 * r + 1
    g1 = np.exp(-((np.arange(s, dtype=np.float64) - r) ** 2) / (2.0 * sig_sq))
    g1 = g1 / g1.sum()
    a = np.zeros((_H, _H), dtype=np.float64)
    for y in range(_H):
        lo = max(0, y - r)
        hi = min(_H, y + r + 1)
        a[y, lo:hi] = g1[(lo - y + r):(hi - y + r)]
    return jnp.asarray(a, dtype=jnp.float32)


def _build_kern(fa_ref, fb_ref, un_ref, k_ref, inv_ref, u_ref, q0_ref):
    fa = fa_ref[0]
    fb = fb_ref[0]
    e = jax.lax.dot_general(fa, fb, (((1,), (1,)), ((), ())),
                            preferred_element_type=jnp.float32)
    k = jnp.exp(jnp.minimum(e, 0.0))
    k_ref[0] = k.astype(jnp.bfloat16)
    rs = jnp.sum(k, axis=1)
    inv_ref[0, 0] = 1.0 / (jnp.sqrt(rs) + 1e-8)
    u = jnp.log(jnp.clip(un_ref[0], 1e-5, 1.0))
    u_ref[0] = u
    m = jnp.max(u, axis=0, keepdims=True)
    ex = jnp.exp(u - m)
    q0_ref[0] = ex / jnp.sum(ex, axis=0, keepdims=True)


def _sconv_kern(q_ref, a_ref, out_ref):
    q = q_ref[0]
    a = a_ref[...]
    s1 = jax.lax.dot_general(q, a, (((1,), (0,)), ((), ())),
                             preferred_element_type=jnp.float32)
    s2 = jax.lax.dot_general(s1, a, (((1,), (0,)), ((), ())),
                             preferred_element_type=jnp.float32)
    out_ref[0] = s2


def _step_kern(q_ref, inv_ref, k_ref, invj_ref, u_ref, qsf_ref, out_ref):
    v = (q_ref[0] * inv_ref[0]).astype(jnp.bfloat16)
    acc = jax.lax.dot_general(v, k_ref[0], (((1,), (0,)), ((), ())),
                              preferred_element_type=jnp.float32)
    qbf = acc * invj_ref[0]
    qh = u_ref[0] + _COMPAT_BF * qbf + _COMPAT_SPATIAL * qsf_ref[0]
    m = jnp.max(qh, axis=0, keepdims=True)
    ex = jnp.exp(qh - m)
    out_ref[0] = ex / jnp.sum(ex, axis=0, keepdims=True)


@jax.jit
def kernel(unary, ref):
    n, c, h, w = unary.shape
    hw = h * w
    kstd = jnp.array([_SXY_BF, _SXY_BF, _SC_BF, _SC_BF, _SC_BF],
                     dtype=jnp.float32)
    y = jax.lax.broadcasted_iota(jnp.float32, (h, w), 0)
    x = jax.lax.broadcasted_iota(jnp.float32, (h, w), 1)
    yx = jnp.broadcast_to(jnp.stack([y, x], 0)[None], (n, 2, h, w))
    f = jnp.concatenate([yx, ref], axis=1) / kstd[None, :, None, None]
    f = f.reshape(n, 5, hw)
    f2 = jnp.sum(f * f, axis=1, keepdims=True)
    ones = jnp.ones_like(f2)
    zeros = jnp.zeros_like(f2)
    fa = jnp.concatenate([f, -0.5 * f2, ones, zeros], axis=1)
    fb = jnp.concatenate([f, ones, -0.5 * f2, zeros], axis=1)
    fa = jnp.transpose(fa, (0, 2, 1))
    fb = jnp.transpose(fb, (0, 2, 1))
    un = unary.reshape(n, c, hw)

    n_i = hw // _I_BLK
    kmat, inv_norm, u, q = pl.pallas_call(
        _build_kern,
        grid=(n, n_i),
        in_specs=[
            pl.BlockSpec((1, _I_BLK, 8), lambda b, i: (b, i, 0)),
            pl.BlockSpec((1, hw, 8), lambda b, i: (b, 0, 0)),
            pl.BlockSpec((1, c, _I_BLK), lambda b, i: (b, 0, i)),
        ],
        out_specs=[
            pl.BlockSpec((1, _I_BLK, hw), lambda b, i: (b, i, 0)),
            pl.BlockSpec((1, 1, _I_BLK), lambda b, i: (b, 0, i)),
            pl.BlockSpec((1, c, _I_BLK), lambda b, i: (b, 0, i)),
            pl.BlockSpec((1, c, _I_BLK), lambda b, i: (b, 0, i)),
        ],
        out_shape=[
            jax.ShapeDtypeStruct((n, hw, hw), jnp.bfloat16),
            jax.ShapeDtypeStruct((n, 1, hw), jnp.float32),
            jax.ShapeDtypeStruct((n, c, hw), jnp.float32),
            jax.ShapeDtypeStruct((n, c, hw), jnp.float32),
        ],
    )(fa, fb, un)

    a = _spatial_matrix()
    sconv = pl.pallas_call(
        _sconv_kern,
        grid=(n,),
        in_specs=[
            pl.BlockSpec((1, c, h, w), lambda b: (b, 0, 0, 0)),
            pl.BlockSpec((h, h), lambda b: (0, 0)),
        ],
        out_specs=pl.BlockSpec((1, c, h, w), lambda b: (b, 0, 0, 0)),
        out_shape=jax.ShapeDtypeStruct((n, c, h, w), jnp.float32),
    )

    n_j = hw // _J_BLK
    step = pl.pallas_call(
        _step_kern,
        grid=(n, n_j),
        in_specs=[
            pl.BlockSpec((1, c, hw), lambda b, j: (b, 0, 0)),
            pl.BlockSpec((1, 1, hw), lambda b, j: (b, 0, 0)),
            pl.BlockSpec((1, hw, _J_BLK), lambda b, j: (b, 0, j)),
            pl.BlockSpec((1, 1, _J_BLK), lambda b, j: (b, 0, j)),
            pl.BlockSpec((1, c, _J_BLK), lambda b, j: (b, 0, j)),
            pl.BlockSpec((1, c, _J_BLK), lambda b, j: (b, 0, j)),
        ],
        out_specs=pl.BlockSpec((1, c, _J_BLK), lambda b, j: (b, 0, j)),
        out_shape=jax.ShapeDtypeStruct((n, c, hw), jnp.float32),
    )

    for _ in range(_NUM_ITER):
        qsf = sconv(q.reshape(n, c, h, w), a).reshape(n, c, hw)
        q = step(q, inv_norm, kmat, inv_norm, u, qsf)
    return q.reshape(n, c, h, w)


# speedup vs baseline: 6.3754x; 1.2031x over previous
import functools

import jax
import jax.numpy as jnp
import numpy as np
from jax.experimental import pallas as pl

_SXY_BF = 70.0
_SC_BF = 12.0
_COMPAT_BF = 4.0
_SXY_SPATIAL = 3
_COMPAT_SPATIAL = 2.0
_NUM_ITER = 2

_H = 64
_W = 64
_HW = _H * _W
_C = 21
_N = 2

_I_BLK = 256
_J_BLK = 1024


def _spatial_matrix():
    sig_sq = float(_SXY_SPATIAL ** 2)
    r = int(sig_sq if sig_sq % 2 else sig_sq - 1)
    s = 2 * r + 1
    g1 = np.exp(-((np.arange(s, dtype=np.float64) - r) ** 2) / (2.0 * sig_sq))
    g1 = g1 / g1.sum()
    a = np.zeros((_H, _H), dtype=np.float64)
    for y in range(_H):
        lo = max(0, y - r)
        hi = min(_H, y + r + 1)
        a[y, lo:hi] = g1[(lo - y + r):(hi - y + r)]
    return jnp.asarray(a, dtype=jnp.float32)


def _build_kern(fa_ref, fb_ref, un_ref, k_ref, inv_ref, u_ref, q0_ref):
    fa = fa_ref[0]
    fb = fb_ref[0]
    e = jax.lax.dot_general(fa, fb, (((1,), (1,)), ((), ())),
                            preferred_element_type=jnp.float32)
    k = jnp.exp(jnp.minimum(e, 0.0))
    k_ref[0] = k.astype(jnp.float8_e4m3fn)
    rs = jnp.sum(k, axis=1)
    inv_ref[0, 0] = 1.0 / (jnp.sqrt(rs) + 1e-8)
    u = jnp.log(jnp.clip(un_ref[0], 1e-5, 1.0))
    u_ref[0] = u
    m = jnp.max(u, axis=0, keepdims=True)
    ex = jnp.exp(u - m)
    q0_ref[0] = ex / jnp.sum(ex, axis=0, keepdims=True)


def _sconv_kern(q_ref, a_ref, out_ref):
    q = q_ref[0]
    a = a_ref[...]
    s1 = jax.lax.dot_general(q, a, (((1,), (0,)), ((), ())),
                             preferred_element_type=jnp.float32)
    s2 = jax.lax.dot_general(s1, a, (((1,), (0,)), ((), ())),
                             preferred_element_type=jnp.float32)
    out_ref[0] = s2


def _step_kern(q_ref, inv_ref, k_ref, invj_ref, u_ref, qsf_ref, out_ref):
    v = (q_ref[0] * inv_ref[0]).astype(jnp.bfloat16)
    acc = jax.lax.dot_general(v, k_ref[0], (((1,), (0,)), ((), ())),
                              preferred_element_type=jnp.float32)
    qbf = acc * invj_ref[0]
    qh = u_ref[0] + _COMPAT_BF * qbf + _COMPAT_SPATIAL * qsf_ref[0]
    m = jnp.max(qh, axis=0, keepdims=True)
    ex = jnp.exp(qh - m)
    out_ref[0] = ex / jnp.sum(ex, axis=0, keepdims=True)


@jax.jit
def kernel(unary, ref):
    n, c, h, w = unary.shape
    hw = h * w
    kstd = jnp.array([_SXY_BF, _SXY_BF, _SC_BF, _SC_BF, _SC_BF],
                     dtype=jnp.float32)
    y = jax.lax.broadcasted_iota(jnp.float32, (h, w), 0)
    x = jax.lax.broadcasted_iota(jnp.float32, (h, w), 1)
    yx = jnp.broadcast_to(jnp.stack([y, x], 0)[None], (n, 2, h, w))
    f = jnp.concatenate([yx, ref], axis=1) / kstd[None, :, None, None]
    f = f.reshape(n, 5, hw)
    f2 = jnp.sum(f * f, axis=1, keepdims=True)
    ones = jnp.ones_like(f2)
    zeros = jnp.zeros_like(f2)
    fa = jnp.concatenate([f, -0.5 * f2, ones, zeros], axis=1)
    fb = jnp.concatenate([f, ones, -0.5 * f2, zeros], axis=1)
    fa = jnp.transpose(fa, (0, 2, 1))
    fb = jnp.transpose(fb, (0, 2, 1))
    un = unary.reshape(n, c, hw)

    n_i = hw // _I_BLK
    kmat, inv_norm, u, q = pl.pallas_call(
        _build_kern,
        grid=(n, n_i),
        in_specs=[
            pl.BlockSpec((1, _I_BLK, 8), lambda b, i: (b, i, 0)),
            pl.BlockSpec((1, hw, 8), lambda b, i: (b, 0, 0)),
            pl.BlockSpec((1, c, _I_BLK), lambda b, i: (b, 0, i)),
        ],
        out_specs=[
            pl.BlockSpec((1, _I_BLK, hw), lambda b, i: (b, i, 0)),
            pl.BlockSpec((1, 1, _I_BLK), lambda b, i: (b, 0, i)),
            pl.BlockSpec((1, c, _I_BLK), lambda b, i: (b, 0, i)),
            pl.BlockSpec((1, c, _I_BLK), lambda b, i: (b, 0, i)),
        ],
        out_shape=[
            jax.ShapeDtypeStruct((n, hw, hw), jnp.float8_e4m3fn),
            jax.ShapeDtypeStruct((n, 1, hw), jnp.float32),
            jax.ShapeDtypeStruct((n, c, hw), jnp.float32),
            jax.ShapeDtypeStruct((n, c, hw), jnp.float32),
        ],
    )(fa, fb, un)

    a = _spatial_matrix()
    sconv = pl.pallas_call(
        _sconv_kern,
        grid=(n,),
        in_specs=[
            pl.BlockSpec((1, c, h, w), lambda b: (b, 0, 0, 0)),
            pl.BlockSpec((h, h), lambda b: (0, 0)),
        ],
        out_specs=pl.BlockSpec((1, c, h, w), lambda b: (b, 0, 0, 0)),
        out_shape=jax.ShapeDtypeStruct((n, c, h, w), jnp.float32),
    )

    n_j = hw // _J_BLK
    step = pl.pallas_call(
        _step_kern,
        grid=(n, n_j),
        in_specs=[
            pl.BlockSpec((1, c, hw), lambda b, j: (b, 0, 0)),
            pl.BlockSpec((1, 1, hw), lambda b, j: (b, 0, 0)),
            pl.BlockSpec((1, hw, _J_BLK), lambda b, j: (b, 0, j)),
            pl.BlockSpec((1, 1, _J_BLK), lambda b, j: (b, 0, j)),
            pl.BlockSpec((1, c, _J_BLK), lambda b, j: (b, 0, j)),
            pl.BlockSpec((1, c, _J_BLK), lambda b, j: (b, 0, j)),
        ],
        out_specs=pl.BlockSpec((1, c, _J_BLK), lambda b, j: (b, 0, j)),
        out_shape=jax.ShapeDtypeStruct((n, c, hw), jnp.float32),
    )

    for _ in range(_NUM_ITER):
        qsf = sconv(q.reshape(n, c, h, w), a).reshape(n, c, hw)
        q = step(q, inv_norm, kmat, inv_norm, u, qsf)
    return q.reshape(n, c, h, w)
